# baseline (speedup 1.0000x reference)
import functools

import jax
import jax.numpy as jnp
from jax.experimental import pallas as pl

_TOP_K = 8


def _row_tile(n, target):
    t = min(n, target)
    while n % t:
        t -= 1
    return t


def _norm_kernel(k_ref, o_ref):
    k = k_ref[0]
    n = jnp.sqrt(jnp.sum(k * k, axis=-1, keepdims=True))
    o_ref[0] = (k / n).astype(jnp.bfloat16)


def _qres_kernel(x_ref, wq_ref, bq_ref, wr_ref, br_ref, q16_ref, res_ref):
    q = jax.lax.dot_general(
        x_ref[...], wq_ref[...], (((1,), (1,)), ((), ())),
        preferred_element_type=jnp.float32) + bq_ref[...]
    q16_ref[...] = q.astype(jnp.bfloat16)
    res_ref[...] = jax.lax.dot_general(
        q, wr_ref[...], (((1,), (1,)), ((), ())),
        preferred_element_type=jnp.float32) + br_ref[...]


def _main_kernel(q_ref, sk_ref, v_ref, res_ref, o_ref):
    q = q_ref[...]
    sk = sk_ref[0]
    scores = jax.lax.dot_general(
        q, sk, (((1,), (1,)), ((), ())),
        preferred_element_type=jnp.float32)
    tr, n = scores.shape
    neg = jnp.float32(-jnp.inf)
    if n % 256 == 0 and n >= 2048:
        ngrp, lvls = 256, 4
    else:
        ngrp, lvls = n, 1
    m3 = scores.reshape(tr, n // ngrp, ngrp)
    cands = []
    for l in range(lvls):
        cur = jnp.max(m3, axis=1, keepdims=True)
        cands.append(cur[:, 0, :])
        if l < lvls - 1:
            m3 = jnp.where(m3 == cur, neg, m3)
    s_top = None
    denom = None
    t8 = None
    for i in range(_TOP_K):
        t = cands[0]
        for a in cands[1:]:
            t = jnp.maximum(t, a)
        cur = jnp.max(t, axis=1, keepdims=True)
        if i == 0:
            s_top = cur
            denom = jnp.ones_like(cur)
        else:
            denom = denom + jnp.exp(cur - s_top)
        t8 = cur
        if i < _TOP_K - 1:
            cands = [jnp.where(a == cur, neg, a) for a in cands]
    ew16 = jnp.where(
        scores >= t8, jnp.exp(scores - s_top), 0.0).astype(jnp.bfloat16)
    out = jax.lax.dot_general(
        ew16, v_ref[0], (((1,), (0,)), ((), ())),
        preferred_element_type=jnp.float32)
    o_ref[0] = out / denom + res_ref[...]


def kernel(input, binding_keys, binding_values, Wq, bq, Wr, br):
    prefix = input.shape[:-1]
    d_in = input.shape[-1]
    n_slots, n_keys, k_dim = binding_keys.shape
    v_dim = binding_values.shape[-1]
    x = input.reshape(-1, d_in)
    bs = x.shape[0]
    values16 = binding_values.astype(jnp.bfloat16)

    sk = pl.pallas_call(
        _norm_kernel,
        grid=(n_slots,),
        in_specs=[pl.BlockSpec((1, n_keys, k_dim), lambda s: (s, 0, 0))],
        out_specs=pl.BlockSpec((1, n_keys, k_dim), lambda s: (s, 0, 0)),
        out_shape=jax.ShapeDtypeStruct((n_slots, n_keys, k_dim), jnp.bfloat16),
    )(binding_keys)

    tq = _row_tile(bs, 512)
    q16, res = pl.pallas_call(
        _qres_kernel,
        grid=(bs // tq,),
        in_specs=[
            pl.BlockSpec((tq, d_in), lambda r: (r, 0)),
            pl.BlockSpec((k_dim, d_in), lambda r: (0, 0)),
            pl.BlockSpec((1, k_dim), lambda r: (0, 0)),
            pl.BlockSpec((v_dim, k_dim), lambda r: (0, 0)),
            pl.BlockSpec((1, v_dim), lambda r: (0, 0)),
        ],
        out_specs=[
            pl.BlockSpec((tq, k_dim), lambda r: (r, 0)),
            pl.BlockSpec((tq, v_dim), lambda r: (r, 0)),
        ],
        out_shape=[
            jax.ShapeDtypeStruct((bs, k_dim), jnp.bfloat16),
            jax.ShapeDtypeStruct((bs, v_dim), jnp.float32),
        ],
    )(x, Wq, bq.reshape(1, -1), Wr, br.reshape(1, -1))

    tr = _row_tile(bs, 512)
    out = pl.pallas_call(
        _main_kernel,
        grid=(n_slots, bs // tr),
        in_specs=[
            pl.BlockSpec((tr, k_dim), lambda s, r: (r, 0)),
            pl.BlockSpec((1, n_keys, k_dim), lambda s, r: (s, 0, 0)),
            pl.BlockSpec((1, n_keys, v_dim), lambda s, r: (s, 0, 0)),
            pl.BlockSpec((tr, v_dim), lambda s, r: (r, 0)),
        ],
        out_specs=pl.BlockSpec((1, tr, v_dim), lambda s, r: (s, r, 0)),
        out_shape=jax.ShapeDtypeStruct((n_slots, bs, v_dim), jnp.float32),
    )(q16, sk, values16, res)

    return jnp.transpose(out, (1, 0, 2)).reshape(prefix + (n_slots, v_dim))


# speedup vs baseline: 1.6304x; 1.6304x over previous
import functools

import jax
import jax.numpy as jnp
from jax.experimental import pallas as pl

_TOP_K = 8


def _row_tile(n, target):
    t = min(n, target)
    while n % t:
        t -= 1
    return t


def _norm_kernel(k_ref, o_ref):
    k = k_ref[0]
    n = jnp.sqrt(jnp.sum(k * k, axis=-1, keepdims=True))
    o_ref[0] = (k / n).astype(jnp.bfloat16)


def _qres_kernel(x_ref, wq_ref, bq_ref, wr_ref, br_ref, q16_ref, res_ref):
    q = jax.lax.dot_general(
        x_ref[...], wq_ref[...], (((1,), (1,)), ((), ())),
        preferred_element_type=jnp.float32) + bq_ref[...]
    q16_ref[...] = q.astype(jnp.bfloat16)
    res_ref[...] = jax.lax.dot_general(
        q, wr_ref[...], (((1,), (1,)), ((), ())),
        preferred_element_type=jnp.float32) + br_ref[...]


def _main_kernel(q_ref, sk_ref, v_ref, res_ref, o_ref):
    q = q_ref[...]
    sk = sk_ref[0]
    scores = jax.lax.dot_general(
        q, sk, (((1,), (1,)), ((), ())),
        preferred_element_type=jnp.float32)
    neg = jnp.float32(-jnp.inf)
    m = scores
    s_top = None
    denom = None
    for i in range(_TOP_K):
        cur = jnp.max(m, axis=1, keepdims=True)
        if i == 0:
            s_top = cur
            denom = jnp.ones_like(cur)
        else:
            denom = denom + jnp.exp(cur - s_top)
        m = jnp.where(m == cur, neg, m)
    ew16 = jnp.where(
        m == neg, jnp.exp(scores - s_top), 0.0).astype(jnp.bfloat16)
    out = jax.lax.dot_general(
        ew16, v_ref[0], (((1,), (0,)), ((), ())),
        preferred_element_type=jnp.float32)
    o_ref[0] = out / denom + res_ref[...]


def kernel(input, binding_keys, binding_values, Wq, bq, Wr, br):
    prefix = input.shape[:-1]
    d_in = input.shape[-1]
    n_slots, n_keys, k_dim = binding_keys.shape
    v_dim = binding_values.shape[-1]
    x = input.reshape(-1, d_in)
    bs = x.shape[0]
    values16 = binding_values.astype(jnp.bfloat16)

    sk = pl.pallas_call(
        _norm_kernel,
        grid=(n_slots,),
        in_specs=[pl.BlockSpec((1, n_keys, k_dim), lambda s: (s, 0, 0))],
        out_specs=pl.BlockSpec((1, n_keys, k_dim), lambda s: (s, 0, 0)),
        out_shape=jax.ShapeDtypeStruct((n_slots, n_keys, k_dim), jnp.bfloat16),
    )(binding_keys)

    tq = _row_tile(bs, 512)
    q16, res = pl.pallas_call(
        _qres_kernel,
        grid=(bs // tq,),
        in_specs=[
            pl.BlockSpec((tq, d_in), lambda r: (r, 0)),
            pl.BlockSpec((k_dim, d_in), lambda r: (0, 0)),
            pl.BlockSpec((1, k_dim), lambda r: (0, 0)),
            pl.BlockSpec((v_dim, k_dim), lambda r: (0, 0)),
            pl.BlockSpec((1, v_dim), lambda r: (0, 0)),
        ],
        out_specs=[
            pl.BlockSpec((tq, k_dim), lambda r: (r, 0)),
            pl.BlockSpec((tq, v_dim), lambda r: (r, 0)),
        ],
        out_shape=[
            jax.ShapeDtypeStruct((bs, k_dim), jnp.bfloat16),
            jax.ShapeDtypeStruct((bs, v_dim), jnp.float32),
        ],
    )(x, Wq, bq.reshape(1, -1), Wr, br.reshape(1, -1))

    tr = _row_tile(bs, 512)
    out = pl.pallas_call(
        _main_kernel,
        grid=(n_slots, bs // tr),
        in_specs=[
            pl.BlockSpec((tr, k_dim), lambda s, r: (r, 0)),
            pl.BlockSpec((1, n_keys, k_dim), lambda s, r: (s, 0, 0)),
            pl.BlockSpec((1, n_keys, v_dim), lambda s, r: (s, 0, 0)),
            pl.BlockSpec((tr, v_dim), lambda s, r: (r, 0)),
        ],
        out_specs=pl.BlockSpec((1, tr, v_dim), lambda s, r: (s, r, 0)),
        out_shape=jax.ShapeDtypeStruct((n_slots, bs, v_dim), jnp.float32),
    )(q16, sk, values16, res)

    return jnp.transpose(out, (1, 0, 2)).reshape(prefix + (n_slots, v_dim))


# speedup vs baseline: 1.6318x; 1.0008x over previous
import jax
import jax.numpy as jnp
from jax.experimental import pallas as pl

_TOP_K = 8


def _row_tile(n, target):
    t = min(n, target)
    while n % t:
        t -= 1
    return t


def _norm_kernel(k_ref, o_ref):
    k = k_ref[0]
    n = jnp.sqrt(jnp.sum(k * k, axis=-1, keepdims=True))
    o_ref[0] = (k / n).astype(jnp.bfloat16)


def _qres_kernel(x_ref, wq_ref, bq_ref, wr_ref, br_ref, q16_ref, res_ref):
    q = jax.lax.dot_general(
        x_ref[...], wq_ref[...], (((1,), (1,)), ((), ())),
        preferred_element_type=jnp.float32) + bq_ref[...]
    q16_ref[...] = q.astype(jnp.bfloat16)
    res_ref[...] = jax.lax.dot_general(
        q, wr_ref[...], (((1,), (1,)), ((), ())),
        preferred_element_type=jnp.float32) + br_ref[...]


def _main_kernel(q_ref, sk_ref, v_ref, res_ref, o_ref):
    q = q_ref[...]
    sk = sk_ref[0]
    scores = jax.lax.dot_general(
        q, sk, (((1,), (1,)), ((), ())),
        preferred_element_type=jnp.float32)
    neg = jnp.float32(-jnp.inf)
    m = scores
    s_top = None
    denom = None
    for i in range(_TOP_K):
        cur = jnp.max(m, axis=1, keepdims=True)
        if i == 0:
            s_top = cur
            denom = jnp.ones_like(cur)
        else:
            denom = denom + jnp.exp(cur - s_top)
        m = jnp.where(m == cur, neg, m)
    ew16 = jnp.where(
        m == neg, jnp.exp(scores - s_top), 0.0).astype(jnp.bfloat16)
    out = jax.lax.dot_general(
        ew16, v_ref[0], (((1,), (0,)), ((), ())),
        preferred_element_type=jnp.float32)
    o_ref[0] = out / denom + res_ref[...]


def kernel(input, binding_keys, binding_values, Wq, bq, Wr, br):
    prefix = input.shape[:-1]
    d_in = input.shape[-1]
    n_slots, n_keys, k_dim = binding_keys.shape
    v_dim = binding_values.shape[-1]
    x = input.reshape(-1, d_in)
    bs = x.shape[0]
    values16 = binding_values.astype(jnp.bfloat16)

    sk = pl.pallas_call(
        _norm_kernel,
        grid=(n_slots,),
        in_specs=[pl.BlockSpec((1, n_keys, k_dim), lambda s: (s, 0, 0))],
        out_specs=pl.BlockSpec((1, n_keys, k_dim), lambda s: (s, 0, 0)),
        out_shape=jax.ShapeDtypeStruct((n_slots, n_keys, k_dim), jnp.bfloat16),
    )(binding_keys)

    tq = _row_tile(bs, 512)
    q16, res = pl.pallas_call(
        _qres_kernel,
        grid=(bs // tq,),
        in_specs=[
            pl.BlockSpec((tq, d_in), lambda r: (r, 0)),
            pl.BlockSpec((k_dim, d_in), lambda r: (0, 0)),
            pl.BlockSpec((1, k_dim), lambda r: (0, 0)),
            pl.BlockSpec((v_dim, k_dim), lambda r: (0, 0)),
            pl.BlockSpec((1, v_dim), lambda r: (0, 0)),
        ],
        out_specs=[
            pl.BlockSpec((tq, k_dim), lambda r: (r, 0)),
            pl.BlockSpec((tq, v_dim), lambda r: (r, 0)),
        ],
        out_shape=[
            jax.ShapeDtypeStruct((bs, k_dim), jnp.bfloat16),
            jax.ShapeDtypeStruct((bs, v_dim), jnp.float32),
        ],
    )(x, Wq, bq.reshape(1, -1), Wr, br.reshape(1, -1))

    tr = _row_tile(bs, 512)
    out = pl.pallas_call(
        _main_kernel,
        grid=(n_slots, bs // tr),
        in_specs=[
            pl.BlockSpec((tr, k_dim), lambda s, r: (r, 0)),
            pl.BlockSpec((1, n_keys, k_dim), lambda s, r: (s, 0, 0)),
            pl.BlockSpec((1, n_keys, v_dim), lambda s, r: (s, 0, 0)),
            pl.BlockSpec((tr, v_dim), lambda s, r: (r, 0)),
        ],
        out_specs=pl.BlockSpec((1, tr, v_dim), lambda s, r: (s, r, 0)),
        out_shape=jax.ShapeDtypeStruct((n_slots, bs, v_dim), jnp.float32),
    )(q16, sk, values16, res)

    return jnp.transpose(out, (1, 0, 2)).reshape(prefix + (n_slots, v_dim))
